# baseline (speedup 1.0000x reference)
import functools

import jax
import jax.numpy as jnp
from jax import lax
from jax.experimental import pallas as pl
from jax.experimental.pallas import tpu as pltpu
from jax.experimental.pallas import tpu_sc as plsc

D_EMB = 64
NC, NS = 2, 16
NW = NC * NS
IDX_MINOR = 128


def _gather_body(idx_hbm, table_hbm, out_hbm, idx_v, rows_v, gsem):
    wid = lax.axis_index("s") * NC + lax.axis_index("c")
    chunks = idx_v.shape[0]
    row_base = wid * chunks
    out_base = row_base * IDX_MINOR

    pltpu.sync_copy(idx_hbm.at[pl.ds(row_base, chunks)], idx_v)

    def body(j, carry):
        pltpu.async_copy(table_hbm.at[idx_v.at[j]], rows_v, gsem).wait()
        pltpu.sync_copy(rows_v, out_hbm.at[pl.ds(out_base + j * IDX_MINOR,
                                                 IDX_MINOR)])
        return carry

    lax.fori_loop(0, chunks, body, 0)


@jax.jit
def kernel(input_ids, table):
    n_tokens = input_ids.shape[0] * input_ids.shape[1]
    assert n_tokens % (NW * IDX_MINOR) == 0
    chunks = n_tokens // (NW * IDX_MINOR)
    idx2d = input_ids.reshape(NW * chunks, IDX_MINOR).astype(jnp.int32)

    mesh = plsc.VectorSubcoreMesh(core_axis_name="c", subcore_axis_name="s",
                                  num_cores=NC, num_subcores=NS)
    run = pl.kernel(
        _gather_body,
        out_type=jax.ShapeDtypeStruct((n_tokens, D_EMB), jnp.float32),
        mesh=mesh,
        scratch_types=[
            pltpu.VMEM((chunks, IDX_MINOR), jnp.int32),
            pltpu.VMEM((IDX_MINOR, D_EMB), jnp.float32),
            pltpu.SemaphoreType.DMA,
        ],
        compiler_params=pltpu.CompilerParams(use_tc_tiling_on_sc=False),
    )
    out = run(idx2d, table)
    return out.reshape(input_ids.shape[0], input_ids.shape[1], D_EMB)


# speedup vs baseline: 1.1101x; 1.1101x over previous
import functools

import jax
import jax.numpy as jnp
from jax import lax
from jax.experimental import pallas as pl
from jax.experimental.pallas import tpu as pltpu
from jax.experimental.pallas import tpu_sc as plsc

D_EMB = 64
NC, NS = 2, 16
NW = NC * NS
IDX_MINOR = 128


K = 4
GROUP = K * IDX_MINOR


def _gather_body(idx_hbm, table_hbm, out_hbm, idx_v, rows_v,
                 gsem0, gsem1, wsem0, wsem1):
    wid = lax.axis_index("s") * NC + lax.axis_index("c")
    chunks = idx_v.shape[0]
    groups = chunks // K
    pairs = groups // 2
    row_base = wid * chunks
    out_base = row_base * IDX_MINOR

    pltpu.sync_copy(idx_hbm.at[pl.ds(row_base, chunks)], idx_v)

    def issue_gathers(g, buf, sem):
        for k in range(K):
            pltpu.async_copy(table_hbm.at[idx_v.at[g * K + k]],
                             rows_v.at[buf, pl.ds(k * IDX_MINOR, IDX_MINOR)],
                             sem)

    def drain(sem, buf):
        pltpu.make_async_copy(out_hbm.at[pl.ds(0, GROUP)],
                              rows_v.at[buf], sem).wait()

    def writeback(g, buf, sem):
        pltpu.async_copy(rows_v.at[buf],
                         out_hbm.at[pl.ds(out_base + g * GROUP, GROUP)], sem)

    issue_gathers(0, 0, gsem0)

    def body(i, carry):
        a = 2 * i
        drain(gsem0, 0)
        writeback(a, 0, wsem0)
        pl.when(i > 0)(lambda: drain(wsem1, 1))
        issue_gathers(a + 1, 1, gsem1)

        def advance():
            drain(wsem0, 0)
            issue_gathers(a + 2, 0, gsem0)
        pl.when(i < pairs - 1)(advance)

        drain(gsem1, 1)
        writeback(a + 1, 1, wsem1)
        return carry

    lax.fori_loop(0, pairs, body, 0)
    drain(wsem0, 0)
    drain(wsem1, 1)


@jax.jit
def kernel(input_ids, table):
    n_tokens = input_ids.shape[0] * input_ids.shape[1]
    assert n_tokens % (NW * IDX_MINOR) == 0
    chunks = n_tokens // (NW * IDX_MINOR)
    idx2d = input_ids.reshape(NW * chunks, IDX_MINOR).astype(jnp.int32)

    mesh = plsc.VectorSubcoreMesh(core_axis_name="c", subcore_axis_name="s",
                                  num_cores=NC, num_subcores=NS)
    run = pl.kernel(
        _gather_body,
        out_type=jax.ShapeDtypeStruct((n_tokens, D_EMB), jnp.float32),
        mesh=mesh,
        scratch_types=[
            pltpu.VMEM((chunks, IDX_MINOR), jnp.int32),
            pltpu.VMEM((2, GROUP, D_EMB), jnp.float32),
            pltpu.SemaphoreType.DMA,
            pltpu.SemaphoreType.DMA,
            pltpu.SemaphoreType.DMA,
            pltpu.SemaphoreType.DMA,
        ],
        compiler_params=pltpu.CompilerParams(use_tc_tiling_on_sc=False),
    )
    out = run(idx2d, table)
    return out.reshape(input_ids.shape[0], input_ids.shape[1], D_EMB)
